# Initial kernel scaffold; baseline (speedup 1.0000x reference)
import functools

import jax
import jax.numpy as jnp
from jax import lax
from jax.experimental import pallas as pl
from jax.experimental.pallas import tpu as pltpu
from jax.experimental.pallas import tpu_sc as plsc

NC = 2
NS = 16
LN = 16


def _make_sc_kernel(B, L, V, D):
    NW = NC * NS
    assert B % NW == 0
    b_per_w = B // NW
    DQ = 16
    NQ = D // DQ
    n_chunks = L // LN

    mesh = plsc.VectorSubcoreMesh(
        core_axis_name="c", subcore_axis_name="s",
        num_cores=NC, num_subcores=NS,
    )

    @functools.partial(
        pl.kernel,
        out_type=jax.ShapeDtypeStruct((B * D * L,), jnp.float32),
        mesh=mesh,
        scratch_types=[
            pltpu.VMEM((L,), jnp.int32),
            pltpu.VMEM((V * D,), jnp.float32),
            pltpu.VMEM((DQ * L,), jnp.float32),
        ],
    )
    def sc_embed(x_hbm, tab_hbm, out_hbm, idx_v, tab_v, obuf):
        wid = lax.axis_index("s") * NC + lax.axis_index("c")
        pltpu.sync_copy(tab_hbm, tab_v)

        def b_body(i, carry):
            b = wid * b_per_w + i
            pltpu.sync_copy(x_hbm.at[pl.ds(b * L, L)], idx_v)
            for q in range(NQ):
                def c_body(cc, carry2):
                    lo = cc * LN
                    base = idx_v[pl.ds(lo, LN)] * D + (q * DQ)
                    for dd in range(DQ):
                        row = plsc.load_gather(tab_v, [base + dd])
                        obuf[pl.ds(dd * L + lo, LN)] = row
                    return carry2
                lax.fori_loop(0, n_chunks, c_body, 0, unroll=2)
                off = (b * D + q * DQ) * L
                pltpu.sync_copy(obuf, out_hbm.at[pl.ds(off, DQ * L)])
            return carry
        lax.fori_loop(0, b_per_w, b_body, 0)

    return sc_embed


def kernel(x, table):
    B, L = x.shape
    V, D = table.shape
    sc_embed = _make_sc_kernel(B, L, V, D)
    out_flat = sc_embed(x.reshape(-1), table.reshape(-1))
    return out_flat.reshape(B, D, L)


# speedup vs baseline: 2.1735x; 2.1735x over previous
import functools

import jax
import jax.numpy as jnp
from jax import lax
from jax.experimental import pallas as pl
from jax.experimental.pallas import tpu as pltpu
from jax.experimental.pallas import tpu_sc as plsc

NC = 2
NS = 16
LN = 16


def _make_sc_kernel(B, L, V, D):
    NW = NC * NS
    assert B % NW == 0
    b_per_w = B // NW
    DQ = 16
    NQ = D // DQ
    n_chunks = L // LN

    mesh = plsc.VectorSubcoreMesh(
        core_axis_name="c", subcore_axis_name="s",
        num_cores=NC, num_subcores=NS,
    )

    @functools.partial(
        pl.kernel,
        out_type=jax.ShapeDtypeStruct((B * D * L,), jnp.float32),
        mesh=mesh,
        compiler_params=pltpu.CompilerParams(needs_layout_passes=False),
        scratch_types=[
            pltpu.VMEM((L,), jnp.int32),
            pltpu.VMEM((V * D,), jnp.float32),
            pltpu.VMEM((DQ * L,), jnp.float32),
        ],
    )
    def sc_embed(x_hbm, tab_hbm, out_hbm, idx_v, tab_v, obuf):
        wid = lax.axis_index("s") * NC + lax.axis_index("c")
        pltpu.sync_copy(tab_hbm, tab_v)

        def b_body(i, carry):
            b = wid * b_per_w + i
            pltpu.sync_copy(x_hbm.at[pl.ds(b * L, L)], idx_v)
            for q in range(NQ):
                def c_body(cc, carry2):
                    lo = cc * LN
                    base = idx_v[pl.ds(lo, LN)] * D + (q * DQ)
                    for dd in range(DQ):
                        row = plsc.load_gather(tab_v, [base + dd])
                        obuf[pl.ds(dd * L + lo, LN)] = row
                    return carry2
                lax.fori_loop(0, n_chunks, c_body, 0, unroll=2)
                off = (b * D + q * DQ) * L
                pltpu.sync_copy(obuf, out_hbm.at[pl.ds(off, DQ * L)])
            return carry
        lax.fori_loop(0, b_per_w, b_body, 0)

    return sc_embed


def kernel(x, table):
    B, L = x.shape
    V, D = table.shape
    sc_embed = _make_sc_kernel(B, L, V, D)
    out_flat = sc_embed(x.reshape(-1), table.reshape(-1))
    return out_flat.reshape(B, D, L)


# speedup vs baseline: 3.2427x; 1.4920x over previous
import functools

import jax
import jax.numpy as jnp
from jax import lax
from jax.experimental import pallas as pl
from jax.experimental.pallas import tpu as pltpu
from jax.experimental.pallas import tpu_sc as plsc

NC = 2
NS = 16
LN = 16


def _make_sc_kernel(B, L, V, D):
    NW = NC * NS
    assert B % NW == 0
    b_per_w = B // NW
    DQ = 16
    NQ = D // DQ
    n_chunks = L // LN

    mesh = plsc.VectorSubcoreMesh(
        core_axis_name="c", subcore_axis_name="s",
        num_cores=NC, num_subcores=NS,
    )

    @functools.partial(
        pl.kernel,
        out_type=jax.ShapeDtypeStruct((B * D * L,), jnp.float32),
        mesh=mesh,
        compiler_params=pltpu.CompilerParams(needs_layout_passes=False),
        scratch_types=[
            pltpu.VMEM((L,), jnp.int32),
            pltpu.VMEM((V * D,), jnp.float32),
            pltpu.VMEM((DQ * L,), jnp.float32),
        ],
    )
    def sc_embed(x_hbm, tab_hbm, out_hbm, idx_v, tab_v, obuf):
        wid = lax.axis_index("s") * NC + lax.axis_index("c")
        pltpu.sync_copy(tab_hbm, tab_v)

        def b_body(i, carry):
            b = wid * b_per_w + i
            pltpu.sync_copy(x_hbm.at[pl.ds(b * L, L)], idx_v)
            for q in range(NQ):
                def c_body(cc, carry2):
                    lo = cc * LN
                    base = idx_v[pl.ds(lo, LN)] * D + (q * DQ)
                    rows = [plsc.load_gather(tab_v, [base + dd])
                            for dd in range(DQ)]
                    for dd in range(DQ):
                        obuf[pl.ds(dd * L + lo, LN)] = rows[dd]
                    return carry2
                lax.fori_loop(0, n_chunks, c_body, 0, unroll=2)
                off = (b * D + q * DQ) * L
                pltpu.sync_copy(obuf, out_hbm.at[pl.ds(off, DQ * L)])
            return carry
        lax.fori_loop(0, b_per_w, b_body, 0)

    return sc_embed


def kernel(x, table):
    B, L = x.shape
    V, D = table.shape
    sc_embed = _make_sc_kernel(B, L, V, D)
    out_flat = sc_embed(x.reshape(-1), table.reshape(-1))
    return out_flat.reshape(B, D, L)


# speedup vs baseline: 3.4531x; 1.0649x over previous
import functools

import jax
import jax.numpy as jnp
from jax import lax
from jax.experimental import pallas as pl
from jax.experimental.pallas import tpu as pltpu
from jax.experimental.pallas import tpu_sc as plsc

NC = 2
NS = 16
LN = 16


def _make_sc_kernel(B, L, V, D):
    NW = NC * NS
    assert B % NW == 0
    b_per_w = B // NW
    DQ = 16
    NQ = D // DQ
    n_chunks = L // LN

    mesh = plsc.VectorSubcoreMesh(
        core_axis_name="c", subcore_axis_name="s",
        num_cores=NC, num_subcores=NS,
    )

    @functools.partial(
        pl.kernel,
        out_type=jax.ShapeDtypeStruct((B * D * L,), jnp.float32),
        mesh=mesh,
        compiler_params=pltpu.CompilerParams(needs_layout_passes=False),
        scratch_types=[
            pltpu.VMEM((L,), jnp.int32),
            pltpu.VMEM((V * D,), jnp.float32),
            pltpu.VMEM((2 * DQ * L,), jnp.float32),
            pltpu.SemaphoreType.DMA,
            pltpu.SemaphoreType.DMA,
        ],
    )
    def sc_embed(x_hbm, tab_hbm, out_hbm, idx_v, tab_v, obuf, sem0, sem1):
        wid = lax.axis_index("s") * NC + lax.axis_index("c")
        sems = [sem0, sem1]
        pltpu.sync_copy(tab_hbm, tab_v)

        def b_body(i, carry):
            b = wid * b_per_w + i
            pltpu.sync_copy(x_hbm.at[pl.ds(b * L, L)], idx_v)
            for q in range(NQ):
                p = q % 2
                buf = obuf.at[pl.ds(p * DQ * L, DQ * L)]

                def drain():
                    pltpu.make_async_copy(
                        buf, out_hbm.at[pl.ds(0, DQ * L)], sems[p]).wait()

                if q >= 2:
                    drain()
                else:
                    pl.when(i > 0)(drain)

                def c_body(cc, carry2):
                    lo = cc * LN
                    base = idx_v[pl.ds(lo, LN)] * D + (q * DQ)
                    rows = [plsc.load_gather(tab_v, [base + dd])
                            for dd in range(DQ)]
                    for dd in range(DQ):
                        buf[pl.ds(dd * L + lo, LN)] = rows[dd]
                    return carry2
                lax.fori_loop(0, n_chunks, c_body, 0, unroll=2)
                off = (b * D + q * DQ) * L
                pltpu.async_copy(buf, out_hbm.at[pl.ds(off, DQ * L)], sems[p])
            return carry
        lax.fori_loop(0, b_per_w, b_body, 0)
        for p in range(2):
            pltpu.make_async_copy(
                obuf.at[pl.ds(p * DQ * L, DQ * L)],
                out_hbm.at[pl.ds(0, DQ * L)], sems[p]).wait()

    return sc_embed


def kernel(x, table):
    B, L = x.shape
    V, D = table.shape
    sc_embed = _make_sc_kernel(B, L, V, D)
    out_flat = sc_embed(x.reshape(-1), table.reshape(-1))
    return out_flat.reshape(B, D, L)


# speedup vs baseline: 8.5773x; 2.4839x over previous
import functools

import jax
import jax.numpy as jnp
from jax import lax
from jax.experimental import pallas as pl
from jax.experimental.pallas import tpu as pltpu
from jax.experimental.pallas import tpu_sc as plsc

NC = 2
NS = 16
LN = 16


def _make_sc_kernel(B, L, V, D):
    NW = NC * NS
    assert B % NW == 0
    b_per_w = B // NW
    DQ = 16
    NQ = D // DQ
    n_chunks = L // LN

    mesh = plsc.VectorSubcoreMesh(
        core_axis_name="c", subcore_axis_name="s",
        num_cores=NC, num_subcores=NS,
    )

    @functools.partial(
        pl.kernel,
        out_type=jax.ShapeDtypeStruct((B * D * L,), jnp.float32),
        mesh=mesh,
        compiler_params=pltpu.CompilerParams(needs_layout_passes=False),
        scratch_types=[
            pltpu.VMEM((L,), jnp.int32),
            pltpu.VMEM((V * D * LN,), jnp.float32),
            pltpu.VMEM((2 * DQ * L,), jnp.float32),
            pltpu.SemaphoreType.DMA,
            pltpu.SemaphoreType.DMA,
        ],
    )
    def sc_embed(x_hbm, tab_hbm, out_hbm, idx_v, tab_v, obuf, sem0, sem1):
        wid = lax.axis_index("s") * NC + lax.axis_index("c")
        sems = [sem0, sem1]
        lane_iota = lax.iota(jnp.int32, LN)
        pltpu.sync_copy(tab_hbm, tab_v)

        def b_body(i, carry):
            b = wid * b_per_w + i
            pltpu.sync_copy(x_hbm.at[pl.ds(b * L, L)], idx_v)
            for q in range(NQ):
                p = q % 2
                buf = obuf.at[pl.ds(p * DQ * L, DQ * L)]

                def drain():
                    pltpu.make_async_copy(
                        buf, out_hbm.at[pl.ds(0, DQ * L)], sems[p]).wait()

                if q >= 2:
                    drain()
                else:
                    pl.when(i > 0)(drain)

                def c_body(cc, carry2):
                    lo = cc * LN
                    base = (idx_v[pl.ds(lo, LN)] * (D * LN)
                            + lane_iota + (q * DQ * LN))
                    rows = [plsc.load_gather(tab_v, [base + dd * LN])
                            for dd in range(DQ)]
                    for dd in range(DQ):
                        buf[pl.ds(dd * L + lo, LN)] = rows[dd]
                    return carry2
                lax.fori_loop(0, n_chunks, c_body, 0, unroll=2)
                off = (b * D + q * DQ) * L
                pltpu.async_copy(buf, out_hbm.at[pl.ds(off, DQ * L)], sems[p])
            return carry
        lax.fori_loop(0, b_per_w, b_body, 0)
        for p in range(2):
            pltpu.make_async_copy(
                obuf.at[pl.ds(p * DQ * L, DQ * L)],
                out_hbm.at[pl.ds(0, DQ * L)], sems[p]).wait()

    return sc_embed


def kernel(x, table):
    B, L = x.shape
    V, D = table.shape
    sc_embed = _make_sc_kernel(B, L, V, D)
    tab_rep = jnp.tile(table.reshape(V * D, 1), (1, LN)).reshape(-1)
    out_flat = sc_embed(x.reshape(-1), tab_rep)
    return out_flat.reshape(B, D, L)


# speedup vs baseline: 8.8226x; 1.0286x over previous
import functools

import jax
import jax.numpy as jnp
from jax import lax
from jax.experimental import pallas as pl
from jax.experimental.pallas import tpu as pltpu
from jax.experimental.pallas import tpu_sc as plsc

NC = 2
NS = 16
LN = 16


def _make_sc_kernel(B, L, V, D):
    NW = NC * NS
    assert B % NW == 0
    b_per_w = B // NW
    DQ = 16
    NQ = D // DQ
    n_chunks = L // LN

    mesh = plsc.VectorSubcoreMesh(
        core_axis_name="c", subcore_axis_name="s",
        num_cores=NC, num_subcores=NS,
    )

    @functools.partial(
        pl.kernel,
        out_type=jax.ShapeDtypeStruct((B * D * L,), jnp.float32),
        mesh=mesh,
        compiler_params=pltpu.CompilerParams(needs_layout_passes=False),
        scratch_types=[
            pltpu.VMEM((L,), jnp.int32),
            pltpu.VMEM((V * D * LN,), jnp.float32),
            pltpu.VMEM((LN,), jnp.int32),
            pltpu.VMEM((2 * DQ * L,), jnp.float32),
            pltpu.SemaphoreType.DMA,
            pltpu.SemaphoreType.DMA,
        ],
    )
    def sc_embed(x_hbm, tab_hbm, iot_hbm, out_hbm, idx_v, tab_v, iot_v, obuf,
                 sem0, sem1):
        wid = lax.axis_index("s") * NC + lax.axis_index("c")
        sems = [sem0, sem1]
        pltpu.sync_copy(iot_hbm, iot_v)
        lane_iota = iot_v[...]
        pltpu.sync_copy(tab_hbm, tab_v)

        def b_body(i, carry):
            b = wid * b_per_w + i
            pltpu.sync_copy(x_hbm.at[pl.ds(b * L, L)], idx_v)
            for q in range(NQ):
                p = q % 2
                buf = obuf.at[pl.ds(p * DQ * L, DQ * L)]

                def drain():
                    pltpu.make_async_copy(
                        buf, out_hbm.at[pl.ds(0, DQ * L)], sems[p]).wait()

                if q >= 2:
                    drain()
                else:
                    pl.when(i > 0)(drain)

                def c_body(cc, carry2):
                    lo = cc * LN
                    base = (idx_v[pl.ds(lo, LN)] * (D * LN)
                            + lane_iota + (q * DQ * LN))
                    rows = [plsc.load_gather(tab_v, [base + dd * LN])
                            for dd in range(DQ)]
                    for dd in range(DQ):
                        buf[pl.ds(dd * L + lo, LN)] = rows[dd]
                    return carry2
                lax.fori_loop(0, n_chunks, c_body, 0, unroll=2)
                off = (b * D + q * DQ) * L
                pltpu.async_copy(buf, out_hbm.at[pl.ds(off, DQ * L)], sems[p])
            return carry
        lax.fori_loop(0, b_per_w, b_body, 0)
        for p in range(2):
            pltpu.make_async_copy(
                obuf.at[pl.ds(p * DQ * L, DQ * L)],
                out_hbm.at[pl.ds(0, DQ * L)], sems[p]).wait()

    return sc_embed


def kernel(x, table):
    B, L = x.shape
    V, D = table.shape
    sc_embed = _make_sc_kernel(B, L, V, D)
    tab_rep = jnp.tile(table.reshape(V * D, 1), (1, LN)).reshape(-1)
    iot = jnp.arange(LN, dtype=jnp.int32)
    out_flat = sc_embed(x.reshape(-1), tab_rep, iot)
    return out_flat.reshape(B, D, L)


# speedup vs baseline: 18.5609x; 2.1038x over previous
import functools

import jax
import jax.numpy as jnp
from jax import lax
from jax.experimental import pallas as pl
from jax.experimental.pallas import tpu as pltpu
from jax.experimental.pallas import tpu_sc as plsc

NC = 2
NS = 16
LN = 16


def _make_sc_kernel(B, L, V, D):
    NW = NC * NS
    assert B % NW == 0
    b_per_w = B // NW
    DQ = 16
    NQ = D // DQ
    n_chunks = L // LN

    mesh = plsc.VectorSubcoreMesh(
        core_axis_name="c", subcore_axis_name="s",
        num_cores=NC, num_subcores=NS,
    )

    @functools.partial(
        pl.kernel,
        out_type=jax.ShapeDtypeStruct((B, D, L), jnp.float32),
        mesh=mesh,
        compiler_params=pltpu.CompilerParams(needs_layout_passes=False),
        scratch_types=[
            pltpu.VMEM((L,), jnp.int32),
            pltpu.VMEM((V * D * LN,), jnp.float32),
            pltpu.VMEM((LN,), jnp.int32),
            pltpu.VMEM((DQ, L), jnp.float32),
            pltpu.VMEM((DQ, L), jnp.float32),
            pltpu.SemaphoreType.DMA,
            pltpu.SemaphoreType.DMA,
        ],
    )
    def sc_embed(x_hbm, tab_hbm, iot_hbm, out_hbm, idx_v, tab_v, iot_v,
                 obuf_a, obuf_b, sem0, sem1):
        wid = lax.axis_index("s") * NC + lax.axis_index("c")
        sems = [sem0, sem1]
        pltpu.sync_copy(iot_hbm, iot_v)
        lane_iota = iot_v[...]
        pltpu.sync_copy(tab_hbm, tab_v)

        def b_body(i, carry):
            b = wid * b_per_w + i
            pltpu.sync_copy(x_hbm.at[pl.ds(b * L, L)], idx_v)
            for q in range(NQ):
                p = q % 2
                buf = (obuf_a, obuf_b)[p]

                def drain():
                    pltpu.make_async_copy(
                        buf, out_hbm.at[0, pl.ds(0, DQ), :], sems[p]).wait()

                if q >= 2:
                    drain()
                else:
                    pl.when(i > 0)(drain)

                def c_body(cc, carry2):
                    lo = cc * LN
                    base = (idx_v[pl.ds(lo, LN)] * (D * LN)
                            + lane_iota + (q * DQ * LN))
                    rows = [plsc.load_gather(tab_v, [base + dd * LN])
                            for dd in range(DQ)]
                    for dd in range(DQ):
                        buf[dd, pl.ds(lo, LN)] = rows[dd]
                    return carry2
                lax.fori_loop(0, n_chunks, c_body, 0, unroll=2)
                pltpu.async_copy(
                    buf, out_hbm.at[b, pl.ds(q * DQ, DQ), :], sems[p])
            return carry
        lax.fori_loop(0, b_per_w, b_body, 0)
        for p in range(2):
            pltpu.make_async_copy(
                (obuf_a, obuf_b)[p],
                out_hbm.at[0, pl.ds(0, DQ), :], sems[p]).wait()

    return sc_embed


def kernel(x, table):
    B, L = x.shape
    V, D = table.shape
    sc_embed = _make_sc_kernel(B, L, V, D)
    tab_rep = jnp.tile(table.reshape(V * D, 1), (1, LN)).reshape(-1)
    iot = jnp.arange(LN, dtype=jnp.int32)
    return sc_embed(x.reshape(-1), tab_rep, iot)


# speedup vs baseline: 19.0571x; 1.0267x over previous
import functools

import jax
import jax.numpy as jnp
from jax import lax
from jax.experimental import pallas as pl
from jax.experimental.pallas import tpu as pltpu
from jax.experimental.pallas import tpu_sc as plsc

NC = 2
NS = 16
LN = 16


def _make_sc_kernel(B, L, V, D):
    NW = NC * NS
    assert B % NW == 0
    b_per_w = B // NW
    DQ = 16
    NQ = D // DQ
    n_chunks = L // LN

    mesh = plsc.VectorSubcoreMesh(
        core_axis_name="c", subcore_axis_name="s",
        num_cores=NC, num_subcores=NS,
    )

    @functools.partial(
        pl.kernel,
        out_type=jax.ShapeDtypeStruct((B, D, L), jnp.float32),
        mesh=mesh,
        compiler_params=pltpu.CompilerParams(needs_layout_passes=False),
        scratch_types=[
            pltpu.VMEM((2 * L,), jnp.int32),
            pltpu.VMEM((V * D * LN,), jnp.float32),
            pltpu.VMEM((LN,), jnp.int32),
            pltpu.VMEM((DQ, L), jnp.float32),
            pltpu.VMEM((DQ, L), jnp.float32),
            pltpu.SemaphoreType.DMA,
            pltpu.SemaphoreType.DMA,
            pltpu.SemaphoreType.DMA,
        ],
    )
    def sc_embed(x_hbm, tab_hbm, iot_hbm, out_hbm, idx_v, tab_v, iot_v,
                 obuf_a, obuf_b, sem0, sem1, sem_idx):
        wid = lax.axis_index("s") * NC + lax.axis_index("c")
        sems = [sem0, sem1]
        pltpu.sync_copy(iot_hbm, iot_v)
        lane_iota = iot_v[...]
        pltpu.sync_copy(tab_hbm, tab_v)
        pltpu.async_copy(x_hbm.at[pl.ds(wid * b_per_w * L, L)],
                         idx_v.at[pl.ds(0, L)], sem_idx)

        def b_body(i, carry):
            b = wid * b_per_w + i
            pbase = (i % 2) * L
            pltpu.make_async_copy(x_hbm.at[pl.ds(0, L)],
                                  idx_v.at[pl.ds(0, L)], sem_idx).wait()
            @pl.when(i + 1 < b_per_w)
            def _prefetch():
                pltpu.async_copy(x_hbm.at[pl.ds((b + 1) * L, L)],
                                 idx_v.at[pl.ds(L - pbase, L)], sem_idx)
            for q in range(NQ):
                p = q % 2
                buf = (obuf_a, obuf_b)[p]

                def drain():
                    pltpu.make_async_copy(
                        buf, out_hbm.at[0, pl.ds(0, DQ), :], sems[p]).wait()

                if q >= 2:
                    drain()
                else:
                    pl.when(i > 0)(drain)

                def c_body(cc, carry2):
                    lo = cc * LN
                    base = (idx_v[pl.ds(pbase + lo, LN)] * (D * LN)
                            + lane_iota + (q * DQ * LN))
                    rows = [plsc.load_gather(tab_v, [base + dd * LN])
                            for dd in range(DQ)]
                    for dd in range(DQ):
                        buf[dd, pl.ds(lo, LN)] = rows[dd]
                    return carry2
                lax.fori_loop(0, n_chunks, c_body, 0, unroll=2)
                pltpu.async_copy(
                    buf, out_hbm.at[b, pl.ds(q * DQ, DQ), :], sems[p])
            return carry
        lax.fori_loop(0, b_per_w, b_body, 0)
        for p in range(2):
            pltpu.make_async_copy(
                (obuf_a, obuf_b)[p],
                out_hbm.at[0, pl.ds(0, DQ), :], sems[p]).wait()

    return sc_embed


def kernel(x, table):
    B, L = x.shape
    V, D = table.shape
    sc_embed = _make_sc_kernel(B, L, V, D)
    tab_rep = jnp.tile(table.reshape(V * D, 1), (1, LN)).reshape(-1)
    iot = jnp.arange(LN, dtype=jnp.int32)
    return sc_embed(x.reshape(-1), tab_rep, iot)
